# Initial kernel scaffold; baseline (speedup 1.0000x reference)
import jax
import jax.numpy as jnp
from jax.experimental import pallas as pl


def kernel(X, edge_index, enc_W, enc_b, dec_W, dec_b, conv_Wl, conv_bl, conv_Wr, gg_Wl, gg_bl, gg_Wr):
    raise NotImplementedError("write your pallas kernel here")


# speedup vs baseline: 2.3800x; 2.3800x over previous
import functools

import jax
import jax.numpy as jnp
from jax import lax
from jax.experimental import pallas as pl
from jax.experimental.pallas import tpu as pltpu
from jax.experimental.pallas import tpu_sc as plsc

_LANES = 16
_NC = 2
_NS = 16
_CHUNK = 128
_HALF = 128


def _ceil_to(x, m):
    return (x + m - 1) // m * m


@functools.cache
def _make_segsum(n_chunks, n_pad, n_wpad):
    cpt = n_chunks // _NS
    zpt = n_pad // (_NS * _CHUNK)
    wpt = n_wpad // _NS
    mesh = plsc.VectorSubcoreMesh(core_axis_name="c", subcore_axis_name="s",
                                  num_cores=_NC, num_subcores=_NS)

    def body(vals, gidx, sidx, out, acc, idxg, idxs, rows):
        c = lax.axis_index("c")
        s = lax.axis_index("s")
        zero = jnp.zeros((_LANES,), jnp.float32)

        @pl.loop(0, _CHUNK)
        def _zero_rows(r):
            for g in range(_HALF // _LANES):
                rows[r, pl.ds(g * _LANES, _LANES)] = zero

        for k in range(zpt):
            pltpu.sync_copy(rows, acc.at[pl.ds((s * zpt + k) * _CHUNK, _CHUNK)])
        plsc.subcore_barrier()

        @pl.loop(0, cpt)
        def _edges(j):
            ch = s * cpt + j
            pltpu.sync_copy(gidx.at[c, ch], idxg)
            pltpu.sync_copy(sidx.at[ch], idxs)
            pltpu.sync_copy(vals.at[idxg.at[0]], rows)
            pltpu.sync_copy(rows, acc.at[idxs.at[0]], add=True)

        plsc.subcore_barrier()
        pltpu.sync_copy(acc.at[pl.ds(s * wpt, wpt)],
                        out.at[c, pl.ds(s * wpt, wpt)])

    return pl.kernel(
        body,
        out_type=jax.ShapeDtypeStruct((_NC, n_wpad, _HALF), jnp.float32),
        mesh=mesh,
        scratch_types=[
            pltpu.VMEM_SHARED((n_pad, _HALF), jnp.float32),
            pltpu.VMEM((1, _CHUNK), jnp.int32),
            pltpu.VMEM((1, _CHUNK), jnp.int32),
            pltpu.VMEM((_CHUNK, _HALF), jnp.float32),
        ],
    )


@functools.cache
def _make_degree(n_chunks, n_pad, n_wpad):
    cpt = n_chunks // _NS
    zpt = n_pad // (_NS * _CHUNK)
    wpt = n_wpad // _NS
    mesh = plsc.VectorSubcoreMesh(core_axis_name="c", subcore_axis_name="s",
                                  num_cores=_NC, num_subcores=_NS)

    def body(didx, out, acc, idxs, buf):
        c = lax.axis_index("c")
        s = lax.axis_index("s")

        @pl.loop(0, _CHUNK)
        def _zero(r):
            for g in range(_HALF // _LANES):
                buf[r, pl.ds(g * _LANES, _LANES)] = jnp.zeros((_LANES,),
                                                              jnp.float32)

        for k in range(zpt):
            pltpu.sync_copy(buf, acc.at[pl.ds((s * zpt + k) * _CHUNK, _CHUNK)])
        plsc.subcore_barrier()

        @pl.loop(0, _CHUNK)
        def _ones(r):
            for g in range(_HALF // _LANES):
                buf[r, pl.ds(g * _LANES, _LANES)] = jnp.ones((_LANES,),
                                                             jnp.float32)

        @pl.loop(0, cpt)
        def _edges(j):
            ch = s * cpt + j
            pltpu.sync_copy(didx.at[c, ch], idxs)
            pltpu.sync_copy(buf, acc.at[idxs.at[0]], add=True)

        plsc.subcore_barrier()
        pltpu.sync_copy(acc.at[pl.ds(s * wpt, wpt)],
                        out.at[c, pl.ds(s * wpt, wpt)])

    return pl.kernel(
        body,
        out_type=jax.ShapeDtypeStruct((_NC, n_wpad, _HALF), jnp.float32),
        mesh=mesh,
        scratch_types=[
            pltpu.VMEM_SHARED((n_pad, _HALF), jnp.float32),
            pltpu.VMEM((1, _CHUNK), jnp.int32),
            pltpu.VMEM((_CHUNK, _HALF), jnp.float32),
        ],
    )


def _pick_bm(n):
    for bm in (512, 400, 256, 200, 128, 80, 40, 8):
        if n % bm == 0:
            return bm
    return n


def _mm(x, w, b, relu):
    n, k = x.shape
    m = w.shape[1]
    bm = _pick_bm(n)

    def body(x_ref, w_ref, b_ref, o_ref):
        acc = jnp.dot(x_ref[...], w_ref[...],
                      preferred_element_type=jnp.float32,
                      precision=lax.Precision.HIGHEST)
        acc = acc + b_ref[...]
        o_ref[...] = jnp.maximum(acc, 0.0) if relu else acc

    return pl.pallas_call(
        body,
        grid=(n // bm,),
        in_specs=[
            pl.BlockSpec((bm, k), lambda i: (i, 0)),
            pl.BlockSpec((k, m), lambda i: (0, 0)),
            pl.BlockSpec((1, m), lambda i: (0, 0)),
        ],
        out_specs=pl.BlockSpec((bm, m), lambda i: (i, 0)),
        out_shape=jax.ShapeDtypeStruct((n, m), jnp.float32),
    )(x, w, b.reshape(1, m))


def _layer_mm(aggsum, deg, h, w_a0, w_a1, w_h, b_all):
    n, d = h.shape
    bm = _pick_bm(n)

    def body(a_ref, c_ref, h_ref, w0_ref, w1_ref, wh_ref, b_ref,
             hn_ref, hg_ref, hq_ref):
        ic = 1.0 / jnp.maximum(c_ref[0][:, 0:1], 1.0)
        acc = (jnp.dot(a_ref[0] * ic, w0_ref[...],
                       preferred_element_type=jnp.float32,
                       precision=lax.Precision.HIGHEST)
               + jnp.dot(a_ref[1] * ic, w1_ref[...],
                         preferred_element_type=jnp.float32,
                         precision=lax.Precision.HIGHEST)
               + jnp.dot(h_ref[...], wh_ref[...],
                         preferred_element_type=jnp.float32,
                         precision=lax.Precision.HIGHEST)
               + b_ref[...])
        hn = jnp.maximum(acc[:, :d], 0.0)
        hg = jnp.maximum(acc[:, d:], 0.0)
        hn_ref[...] = hn
        hg_ref[...] = hg
        hq_ref[...] = hg * hg

    sds = jax.ShapeDtypeStruct((n, d), jnp.float32)
    return pl.pallas_call(
        body,
        grid=(n // bm,),
        in_specs=[
            pl.BlockSpec((_NC, bm, _HALF), lambda i: (0, i, 0)),
            pl.BlockSpec((1, bm, _HALF), lambda i: (1, i, 0)),
            pl.BlockSpec((bm, d), lambda i: (i, 0)),
            pl.BlockSpec((_HALF, 2 * d), lambda i: (0, 0)),
            pl.BlockSpec((_HALF, 2 * d), lambda i: (0, 0)),
            pl.BlockSpec((d, 2 * d), lambda i: (0, 0)),
            pl.BlockSpec((1, 2 * d), lambda i: (0, 0)),
        ],
        out_specs=[
            pl.BlockSpec((bm, d), lambda i: (i, 0)),
            pl.BlockSpec((bm, d), lambda i: (i, 0)),
            pl.BlockSpec((bm, d), lambda i: (i, 0)),
        ],
        out_shape=[sds, sds, sds],
    )(aggsum, deg, h, w_a0, w_a1, w_h, b_all.reshape(1, 2 * d))


def _gate(h, hn, hg, s1, s2, deg):
    n, d = h.shape
    bm = _pick_bm(n)

    def body(h_ref, hn_ref, hg_ref, s1_ref, s2_ref, d_ref, o_ref):
        dd = d_ref[0][:, 0:1]
        invd = 1.0 / jnp.maximum(dd, 1.0)
        hgv = hg_ref[...]
        s1v = jnp.concatenate([s1_ref[0], s1_ref[1]], axis=1)
        s2v = jnp.concatenate([s2_ref[0], s2_ref[1]], axis=1)
        num = dd * hgv * hgv - 2.0 * hgv * s1v + s2v
        tau = jnp.tanh(num * invd)
        hv = h_ref[...]
        o_ref[...] = hv + tau * (hn_ref[...] - hv)

    return pl.pallas_call(
        body,
        grid=(n // bm,),
        in_specs=[
            pl.BlockSpec((bm, d), lambda i: (i, 0)),
            pl.BlockSpec((bm, d), lambda i: (i, 0)),
            pl.BlockSpec((bm, d), lambda i: (i, 0)),
            pl.BlockSpec((_NC, bm, _HALF), lambda i: (0, i, 0)),
            pl.BlockSpec((_NC, bm, _HALF), lambda i: (0, i, 0)),
            pl.BlockSpec((1, bm, _HALF), lambda i: (0, i, 0)),
        ],
        out_specs=pl.BlockSpec((bm, d), lambda i: (i, 0)),
        out_shape=jax.ShapeDtypeStruct((n, d), jnp.float32),
    )(h, hn, hg, s1, s2, deg)


def kernel(X, edge_index, enc_W, enc_b, dec_W, dec_b,
           conv_Wl, conv_bl, conv_Wr, gg_Wl, gg_bl, gg_Wr):
    n = X.shape[0]
    e = edge_index.shape[1]
    d = conv_Wl.shape[0]

    n_chunks = _ceil_to((e + _CHUNK - 1) // _CHUNK, _NS)
    e_pad = n_chunks * _CHUNK
    n_pad = _ceil_to(n + 1, _NS * _CHUNK)
    n_wpad = _ceil_to(n, _NS * 8)

    src = edge_index[0]
    dst = edge_index[1]
    padz = jnp.zeros((e_pad - e,), jnp.int32)
    padt = jnp.full((e_pad - e,), n, jnp.int32)
    srcg = jnp.concatenate([src, padz])
    dstg = jnp.concatenate([dst, padz])
    srcs = jnp.concatenate([src, padt]).reshape(n_chunks, 1, _CHUNK)
    dsts = jnp.concatenate([dst, padt]).reshape(n_chunks, 1, _CHUNK)
    gsrc = jnp.stack([2 * srcg, 2 * srcg + 1]).reshape(_NC, n_chunks, 1, _CHUNK)
    gdst = jnp.stack([2 * dstg, 2 * dstg + 1]).reshape(_NC, n_chunks, 1, _CHUNK)
    didx = jnp.stack([srcs, dsts])

    seg = _make_segsum(n_chunks, n_pad, n_wpad)
    degk = _make_degree(n_chunks, n_pad, n_wpad)

    deg = degk(didx)
    H = _mm(X, enc_W, enc_b, True)

    w_conv = jnp.concatenate([conv_Wl, gg_Wl], axis=1)
    w_a0 = w_conv[:_HALF]
    w_a1 = w_conv[_HALF:]
    w_h = jnp.concatenate([conv_Wr, gg_Wr], axis=1)
    b_all = jnp.concatenate([conv_bl, gg_bl])

    for _ in range(3):
        aggsum = seg(H.reshape(2 * n, _HALF), gsrc, dsts)
        hn, hg, hq = _layer_mm(aggsum, deg, H, w_a0, w_a1, w_h, b_all)
        s1 = seg(hg.reshape(2 * n, _HALF), gdst, srcs)
        s2 = seg(hq.reshape(2 * n, _HALF), gdst, srcs)
        H = _gate(H, hn, hg, s1, s2, deg)

    return _mm(H, dec_W, dec_b, False)
